# baseline (speedup 1.0000x reference)
import functools

import jax
import jax.numpy as jnp
from jax.experimental import pallas as pl
from jax.experimental.pallas import tpu as pltpu

PROJ_WIDTH = 128
BASE = 10000.0
SEQ = 4096
BS = 1024


def _rope_body(sid_ref, x_ref, o_ref, cos_ref, sin_ref):
    h = pl.program_id(2)

    @pl.when(h == 0)
    def _compute_trig():
        sid = sid_ref[0, 0, :].astype(jnp.float32)
        d = jax.lax.broadcasted_iota(jnp.int32, (BS, PROJ_WIDTH), 1)
        expnt = ((d // 2) * 2).astype(jnp.float32) * (1.0 / PROJ_WIDTH)
        theta = jnp.exp(-jnp.log(BASE) * expnt)
        angle = sid[:, None] * theta
        cos_ref[...] = jnp.cos(angle)
        sg = jnp.where((d % 2) == 0, -1.0, 1.0)
        sin_ref[...] = sg * jnp.sin(angle)

    xb = x_ref[0, 0]
    c = cos_ref[...]
    s = sin_ref[...]
    d = jax.lax.broadcasted_iota(jnp.int32, (BS, PROJ_WIDTH), 1)
    even = (d % 2) == 0
    swp = jnp.where(even, pltpu.roll(xb, PROJ_WIDTH - 1, 1), pltpu.roll(xb, 1, 1))
    o_ref[0, 0] = c * xb + s * swp


@jax.jit
def kernel(x, seq_id):
    B, H1, H2, S, W = x.shape
    H = H1 * H2
    n_sblk = S // BS
    xr = x.reshape(B, H, S, W)
    sid = seq_id.reshape(B * n_sblk, 1, BS)

    out = pl.pallas_call(
        _rope_body,
        grid=(B, n_sblk, H),
        in_specs=[
            pl.BlockSpec((1, 1, BS), lambda b, sblk, h: (b * n_sblk + sblk, 0, 0)),
            pl.BlockSpec((1, 1, BS, W), lambda b, sblk, h: (b, h, sblk, 0)),
        ],
        out_specs=pl.BlockSpec((1, 1, BS, W), lambda b, sblk, h: (b, h, sblk, 0)),
        out_shape=jax.ShapeDtypeStruct((B, H, S, W), jnp.float32),
        scratch_shapes=[
            pltpu.VMEM((BS, W), jnp.float32),
            pltpu.VMEM((BS, W), jnp.float32),
        ],
    )(sid, xr)
    return out.reshape(B, H1, H2, S, W)


# speedup vs baseline: 1.3585x; 1.3585x over previous
import functools

import jax
import jax.numpy as jnp
from jax.experimental import pallas as pl
from jax.experimental.pallas import tpu as pltpu

PROJ_WIDTH = 128
BASE = 10000.0
SEQ = 4096
BS = 2048


def _rope_body(sid_ref, x_ref, o_ref, cos_ref, sin_ref):
    h = pl.program_id(2)

    @pl.when(h == 0)
    def _compute_trig():
        sid = sid_ref[0, 0, :].astype(jnp.float32)
        d = jax.lax.broadcasted_iota(jnp.int32, (BS, PROJ_WIDTH), 1)
        expnt = ((d // 2) * 2).astype(jnp.float32) * (1.0 / PROJ_WIDTH)
        theta = jnp.exp(-jnp.log(BASE) * expnt)
        angle = sid[:, None] * theta
        cos_ref[...] = jnp.cos(angle)
        sg = jnp.where((d % 2) == 0, -1.0, 1.0)
        sin_ref[...] = sg * jnp.sin(angle)

    xb = x_ref[0, 0]
    c = cos_ref[...]
    s = sin_ref[...]
    d = jax.lax.broadcasted_iota(jnp.int32, (BS, PROJ_WIDTH), 1)
    even = (d % 2) == 0
    swp = jnp.where(even, pltpu.roll(xb, PROJ_WIDTH - 1, 1), pltpu.roll(xb, 1, 1))
    o_ref[0, 0] = c * xb + s * swp


@jax.jit
def kernel(x, seq_id):
    B, H1, H2, S, W = x.shape
    H = H1 * H2
    n_sblk = S // BS
    xr = x.reshape(B, H, S, W)
    sid = seq_id.reshape(B * n_sblk, 1, BS)

    out = pl.pallas_call(
        _rope_body,
        grid=(B, n_sblk, H),
        in_specs=[
            pl.BlockSpec((1, 1, BS), lambda b, sblk, h: (b * n_sblk + sblk, 0, 0)),
            pl.BlockSpec((1, 1, BS, W), lambda b, sblk, h: (b, h, sblk, 0)),
        ],
        out_specs=pl.BlockSpec((1, 1, BS, W), lambda b, sblk, h: (b, h, sblk, 0)),
        out_shape=jax.ShapeDtypeStruct((B, H, S, W), jnp.float32),
        scratch_shapes=[
            pltpu.VMEM((BS, W), jnp.float32),
            pltpu.VMEM((BS, W), jnp.float32),
        ],
    )(sid, xr)
    return out.reshape(B, H1, H2, S, W)


# speedup vs baseline: 1.7443x; 1.2840x over previous
import functools

import jax
import jax.numpy as jnp
from jax.experimental import pallas as pl
from jax.experimental.pallas import tpu as pltpu

PROJ_WIDTH = 128
BASE = 10000.0
SEQ = 4096
BS = 4096


def _rope_body(sid_ref, x_ref, o_ref, cos_ref, sin_ref):
    h = pl.program_id(2)

    @pl.when(h == 0)
    def _compute_trig():
        sid = sid_ref[0, 0, :].astype(jnp.float32)
        d = jax.lax.broadcasted_iota(jnp.int32, (BS, PROJ_WIDTH), 1)
        expnt = ((d // 2) * 2).astype(jnp.float32) * (1.0 / PROJ_WIDTH)
        theta = jnp.exp(-jnp.log(BASE) * expnt)
        angle = sid[:, None] * theta
        cos_ref[...] = jnp.cos(angle)
        sg = jnp.where((d % 2) == 0, -1.0, 1.0)
        sin_ref[...] = sg * jnp.sin(angle)

    xb = x_ref[0, 0]
    c = cos_ref[...]
    s = sin_ref[...]
    d = jax.lax.broadcasted_iota(jnp.int32, (BS, PROJ_WIDTH), 1)
    even = (d % 2) == 0
    swp = jnp.where(even, pltpu.roll(xb, PROJ_WIDTH - 1, 1), pltpu.roll(xb, 1, 1))
    o_ref[0, 0] = c * xb + s * swp


@jax.jit
def kernel(x, seq_id):
    B, H1, H2, S, W = x.shape
    H = H1 * H2
    n_sblk = S // BS
    xr = x.reshape(B, H, S, W)
    sid = seq_id.reshape(B * n_sblk, 1, BS)

    out = pl.pallas_call(
        _rope_body,
        grid=(B, n_sblk, H),
        in_specs=[
            pl.BlockSpec((1, 1, BS), lambda b, sblk, h: (b * n_sblk + sblk, 0, 0)),
            pl.BlockSpec((1, 1, BS, W), lambda b, sblk, h: (b, h, sblk, 0)),
        ],
        out_specs=pl.BlockSpec((1, 1, BS, W), lambda b, sblk, h: (b, h, sblk, 0)),
        out_shape=jax.ShapeDtypeStruct((B, H, S, W), jnp.float32),
        scratch_shapes=[
            pltpu.VMEM((BS, W), jnp.float32),
            pltpu.VMEM((BS, W), jnp.float32),
        ],
    )(sid, xr)
    return out.reshape(B, H1, H2, S, W)


# speedup vs baseline: 2.4567x; 1.4084x over previous
import functools

import jax
import jax.numpy as jnp
from jax.experimental import pallas as pl
from jax.experimental.pallas import tpu as pltpu

PROJ_WIDTH = 128
BASE = 10000.0
SEQ = 4096
BS = 4096
H_BLK = 4


def _rope_body(sid_ref, x_ref, o_ref, cos_ref, sin_ref):
    h = pl.program_id(2)

    @pl.when(h == 0)
    def _compute_trig():
        sid = sid_ref[0, 0, :].astype(jnp.float32)
        d = jax.lax.broadcasted_iota(jnp.int32, (BS, PROJ_WIDTH), 1)
        expnt = ((d // 2) * 2).astype(jnp.float32) * (1.0 / PROJ_WIDTH)
        theta = jnp.exp(-jnp.log(BASE) * expnt)
        angle = sid[:, None] * theta
        cos_ref[...] = jnp.cos(angle)
        sg = jnp.where((d % 2) == 0, -1.0, 1.0)
        sin_ref[...] = sg * jnp.sin(angle)

    xb = x_ref[0]
    c = cos_ref[...][None]
    s = sin_ref[...][None]
    d = jax.lax.broadcasted_iota(jnp.int32, (1, BS, PROJ_WIDTH), 2)
    even = (d % 2) == 0
    swp = jnp.where(even, pltpu.roll(xb, PROJ_WIDTH - 1, 2), pltpu.roll(xb, 1, 2))
    o_ref[0] = c * xb + s * swp


@jax.jit
def kernel(x, seq_id):
    B, H1, H2, S, W = x.shape
    H = H1 * H2
    n_sblk = S // BS
    xr = x.reshape(B, H, S, W)
    sid = seq_id.reshape(B * n_sblk, 1, BS)

    out = pl.pallas_call(
        _rope_body,
        grid=(B, n_sblk, H // H_BLK),
        in_specs=[
            pl.BlockSpec((1, 1, BS), lambda b, sblk, h: (b * n_sblk + sblk, 0, 0)),
            pl.BlockSpec((1, H_BLK, BS, W), lambda b, sblk, h: (b, h, sblk, 0)),
        ],
        out_specs=pl.BlockSpec((1, H_BLK, BS, W), lambda b, sblk, h: (b, h, sblk, 0)),
        out_shape=jax.ShapeDtypeStruct((B, H, S, W), jnp.float32),
        scratch_shapes=[
            pltpu.VMEM((BS, W), jnp.float32),
            pltpu.VMEM((BS, W), jnp.float32),
        ],
    )(sid, xr)
    return out.reshape(B, H1, H2, S, W)
